# baseline (speedup 1.0000x reference)
import functools

import jax
import jax.numpy as jnp
from jax import lax
from jax.experimental import pallas as pl
from jax.experimental.pallas import tpu as pltpu
from jax.experimental.pallas import tpu_sc as plsc

_VOCAB = 1000000
_D = 64
_B = 16384
_NEG = 5

_NW = 32
_RW = _B // _NW
_CH = 128
_NCH = _RW // _CH
_NG = _CH // 16

_LN2 = 0.6931471805599453
_SQRT2 = 1.4142135623730951


def _log_f32(y):
    bits = plsc.bitcast(y, jnp.int32)
    e = ((bits >> 23) & 0xFF) - 127
    m = plsc.bitcast((bits & 0x7FFFFF) | (127 << 23), jnp.float32)
    big = m > _SQRT2
    m = jnp.where(big, m * 0.5, m)
    e = e + big.astype(jnp.int32)
    r = (m - 1.0) / (m + 1.0)
    r2 = r * r
    p = r2 * (1.0 / 9.0) + (1.0 / 7.0)
    p = p * r2 + (1.0 / 5.0)
    p = p * r2 + (1.0 / 3.0)
    p = p * r2 + 1.0
    return e.astype(jnp.float32) * _LN2 + 2.0 * r * p


def _softplus(x):
    return _log_f32(1.0 + jnp.exp(x))


def _sc_partials(pos_u, pos_v, neg_t, u_pairs, v_pairs):
    mesh = plsc.VectorSubcoreMesh(core_axis_name="c", subcore_axis_name="s")

    @functools.partial(
        pl.kernel,
        mesh=mesh,
        out_type=jax.ShapeDtypeStruct((_NW, 16), jnp.float32),
        compiler_params=pltpu.CompilerParams(needs_layout_passes=False),
        scratch_types=[
            pltpu.VMEM((_CH,), jnp.int32),
            pltpu.VMEM((_CH,), jnp.int32),
            pltpu.VMEM((_NEG, _CH), jnp.int32),
            pltpu.VMEM((_CH,), jnp.int32),
            pltpu.VMEM((_CH,), jnp.int32),
            pltpu.VMEM((_NEG, _CH), jnp.int32),
            pltpu.VMEM((_CH, 128), jnp.float32),
            pltpu.VMEM((_CH, 128), jnp.float32),
            pltpu.VMEM((_NEG * _CH, 128), jnp.float32),
            pltpu.VMEM((16,), jnp.float32),
            pltpu.SemaphoreType.DMA,
        ],
    )
    def k(pu_hbm, pv_hbm, nt_hbm, uw_hbm, vw_hbm, out_hbm,
          raw_u, raw_v, raw_n, pair_u, pair_v, pair_n,
          u_buf, v_buf, n_buf, accv, sem):
        wid = lax.axis_index("s") * 2 + lax.axis_index("c")
        row0 = wid * _RW

        def chunk_body(ci, acc):
            base = row0 + ci * _CH
            pltpu.sync_copy(pu_hbm.at[pl.ds(base, _CH)], raw_u)
            pltpu.sync_copy(pv_hbm.at[pl.ds(base, _CH)], raw_v)
            for j in range(_NEG):
                pltpu.sync_copy(nt_hbm.at[pl.ds(j, 1), pl.ds(base, _CH)],
                                raw_n.at[pl.ds(j, 1)])

            def halve_body(i, _):
                sl = pl.ds(i * 16, 16)
                pair_u[sl] = raw_u[sl] >> 1
                pair_v[sl] = raw_v[sl] >> 1
                for j in range(_NEG):
                    pair_n[j, sl] = raw_n[j, sl] >> 1
                return 0
            lax.fori_loop(0, _CH // 16, halve_body, 0)

            cp_u = pltpu.async_copy(uw_hbm.at[pair_u], u_buf, sem)
            cp_v = pltpu.async_copy(vw_hbm.at[pair_v], v_buf, sem)
            cps = [pltpu.async_copy(vw_hbm.at[pair_n.at[j]],
                                    n_buf.at[pl.ds(j * _CH, _CH)], sem)
                   for j in range(_NEG)]
            cp_u.wait()
            cp_v.wait()
            for cp in cps:
                cp.wait()

            def group_body(g, acc):
                rows = g * 16 + lax.iota(jnp.int32, 16)
                cb_u = (raw_u[pl.ds(g * 16, 16)] & 1) * 64
                cb_v = (raw_v[pl.ds(g * 16, 16)] & 1) * 64
                cb_n = [(raw_n[j, pl.ds(g * 16, 16)] & 1) * 64
                        for j in range(_NEG)]
                pn = [rows + j * _CH for j in range(_NEG)]

                def d_body(d, carry):
                    sp, s0, s1, s2, s3, s4 = carry
                    du = plsc.load_gather(u_buf, [rows, cb_u + d])
                    dv = plsc.load_gather(v_buf, [rows, cb_v + d])
                    sp = sp + du * dv
                    n0 = plsc.load_gather(n_buf, [pn[0], cb_n[0] + d])
                    s0 = s0 + du * n0
                    n1 = plsc.load_gather(n_buf, [pn[1], cb_n[1] + d])
                    s1 = s1 + du * n1
                    n2 = plsc.load_gather(n_buf, [pn[2], cb_n[2] + d])
                    s2 = s2 + du * n2
                    n3 = plsc.load_gather(n_buf, [pn[3], cb_n[3] + d])
                    s3 = s3 + du * n3
                    n4 = plsc.load_gather(n_buf, [pn[4], cb_n[4] + d])
                    s4 = s4 + du * n4
                    return (sp, s0, s1, s2, s3, s4)

                z = jnp.zeros((16,), jnp.float32)
                sp, s0, s1, s2, s3, s4 = lax.fori_loop(
                    0, _D, d_body, (z, z, z, z, z, z))
                val = _softplus(-jnp.clip(sp, -10.0, 10.0))
                for sk in (s0, s1, s2, s3, s4):
                    val = val + _softplus(jnp.clip(sk, -10.0, 10.0))
                return acc + val

            return lax.fori_loop(0, _NG, group_body, acc)

        acc = lax.fori_loop(0, _NCH, chunk_body, jnp.zeros((16,), jnp.float32))
        accv[...] = acc
        pltpu.sync_copy(accv, out_hbm.at[wid])

    return k(pos_u, pos_v, neg_t, u_pairs, v_pairs)


_RW_BLK = 1024


def _repack(w_t):
    def body(in_ref, o_ref):
        r = lax.broadcasted_iota(jnp.int32, (64, 64), 0)
        c = lax.broadcasted_iota(jnp.int32, (64, 64), 1)
        eye = (r == c).astype(jnp.float32)
        x = in_ref[...]
        y = jax.lax.dot_general(x, eye, (((0,), (0,)), ((), ())),
                                precision=jax.lax.Precision.HIGHEST)
        y3 = y.reshape(_RW_BLK, 2, 64)
        o_ref[...] = jnp.concatenate([y3[:, 0, :], y3[:, 1, :]], axis=1)

    return pl.pallas_call(
        body,
        grid=(pl.cdiv(_VOCAB // 2, _RW_BLK),),
        in_specs=[pl.BlockSpec((64, 2 * _RW_BLK), lambda j: (0, j))],
        out_specs=pl.BlockSpec((_RW_BLK, 128), lambda j: (j, 0)),
        out_shape=jax.ShapeDtypeStruct((_VOCAB // 2, 128), jnp.float32),
    )(w_t)


def _finalize(partials):
    def body(p_ref, o_ref):
        o_ref[0, 0] = jnp.sum(p_ref[...]) * (1.0 / _B)

    out = pl.pallas_call(
        body,
        out_shape=jax.ShapeDtypeStruct((1, 1), jnp.float32),
        out_specs=pl.BlockSpec(memory_space=pltpu.SMEM),
    )(partials)
    return out[0, 0]


def kernel(pos_u, pos_v, neg_v, u_weight, v_weight):
    u_pairs = _repack(u_weight.T)
    v_pairs = _repack(v_weight.T)
    neg_t = neg_v.astype(jnp.int32).T
    partials = _sc_partials(pos_u.astype(jnp.int32), pos_v.astype(jnp.int32),
                            neg_t, u_pairs, v_pairs)
    return _finalize(partials)


# speedup vs baseline: 2.1118x; 2.1118x over previous
import functools

import jax
import jax.numpy as jnp
from jax import lax
from jax.experimental import pallas as pl
from jax.experimental.pallas import tpu as pltpu
from jax.experimental.pallas import tpu_sc as plsc

_VOCAB = 1000000
_D = 64
_B = 16384
_NEG = 5

_NW = 32
_RW = _B // _NW
_CH = 128
_NCH = _RW // _CH
_NG = _CH // 16

_LN2 = 0.6931471805599453
_SQRT2 = 1.4142135623730951


def _log_f32(y):
    bits = plsc.bitcast(y, jnp.int32)
    e = ((bits >> 23) & 0xFF) - 127
    m = plsc.bitcast((bits & 0x7FFFFF) | (127 << 23), jnp.float32)
    big = m > _SQRT2
    m = jnp.where(big, m * 0.5, m)
    e = e + big.astype(jnp.int32)
    r = (m - 1.0) / (m + 1.0)
    r2 = r * r
    p = r2 * (1.0 / 9.0) + (1.0 / 7.0)
    p = p * r2 + (1.0 / 5.0)
    p = p * r2 + (1.0 / 3.0)
    p = p * r2 + 1.0
    return e.astype(jnp.float32) * _LN2 + 2.0 * r * p


def _softplus(x):
    return _log_f32(1.0 + jnp.exp(x))


def _sc_partials(pos_u, pos_v, neg_t, u_pairs, v_pairs):
    mesh = plsc.VectorSubcoreMesh(core_axis_name="c", subcore_axis_name="s")

    @functools.partial(
        pl.kernel,
        mesh=mesh,
        out_type=jax.ShapeDtypeStruct((_NW, 16), jnp.float32),
        compiler_params=pltpu.CompilerParams(needs_layout_passes=False),
        scratch_types=[
            pltpu.VMEM((_CH,), jnp.int32),
            pltpu.VMEM((_CH,), jnp.int32),
            pltpu.VMEM((_NEG, _CH), jnp.int32),
            pltpu.VMEM((_CH,), jnp.int32),
            pltpu.VMEM((_CH,), jnp.int32),
            pltpu.VMEM((_NEG, _CH), jnp.int32),
            pltpu.VMEM((_CH, 128), jnp.float32),
            pltpu.VMEM((_CH, 128), jnp.float32),
            pltpu.VMEM((_NEG * _CH, 128), jnp.float32),
            pltpu.VMEM((16,), jnp.float32),
            pltpu.SemaphoreType.DMA,
        ],
    )
    def k(pu_hbm, pv_hbm, nt_hbm, uw_hbm, vw_hbm, out_hbm,
          raw_u, raw_v, raw_n, pair_u, pair_v, pair_n,
          u_buf, v_buf, n_buf, accv, sem):
        wid = lax.axis_index("s") * 2 + lax.axis_index("c")
        row0 = wid * _RW

        def chunk_body(ci, acc):
            base = row0 + ci * _CH
            pltpu.sync_copy(pu_hbm.at[pl.ds(base, _CH)], raw_u)
            pltpu.sync_copy(pv_hbm.at[pl.ds(base, _CH)], raw_v)
            for j in range(_NEG):
                pltpu.sync_copy(nt_hbm.at[pl.ds(j, 1), pl.ds(base, _CH)],
                                raw_n.at[pl.ds(j, 1)])

            def _packed_row(r):
                return ((r >> 12) << 11) + (r & 2047)

            def halve_body(i, _):
                sl = pl.ds(i * 16, 16)
                pair_u[sl] = _packed_row(raw_u[sl])
                pair_v[sl] = _packed_row(raw_v[sl])
                for j in range(_NEG):
                    pair_n[j, sl] = _packed_row(raw_n[j, sl])
                return 0
            lax.fori_loop(0, _CH // 16, halve_body, 0)

            cp_u = pltpu.async_copy(uw_hbm.at[pair_u], u_buf, sem)
            cp_v = pltpu.async_copy(vw_hbm.at[pair_v], v_buf, sem)
            cps = [pltpu.async_copy(vw_hbm.at[pair_n.at[j]],
                                    n_buf.at[pl.ds(j * _CH, _CH)], sem)
                   for j in range(_NEG)]
            cp_u.wait()
            cp_v.wait()
            for cp in cps:
                cp.wait()

            def group_body(g, acc):
                rows = g * 16 + lax.iota(jnp.int32, 16)
                cb_u = ((raw_u[pl.ds(g * 16, 16)] >> 11) & 1) * 64
                cb_v = ((raw_v[pl.ds(g * 16, 16)] >> 11) & 1) * 64
                cb_n = [((raw_n[j, pl.ds(g * 16, 16)] >> 11) & 1) * 64
                        for j in range(_NEG)]
                pn = [rows + j * _CH for j in range(_NEG)]

                def d_body(d, carry):
                    sp, s0, s1, s2, s3, s4 = carry
                    du = plsc.load_gather(u_buf, [rows, cb_u + d])
                    dv = plsc.load_gather(v_buf, [rows, cb_v + d])
                    sp = sp + du * dv
                    n0 = plsc.load_gather(n_buf, [pn[0], cb_n[0] + d])
                    s0 = s0 + du * n0
                    n1 = plsc.load_gather(n_buf, [pn[1], cb_n[1] + d])
                    s1 = s1 + du * n1
                    n2 = plsc.load_gather(n_buf, [pn[2], cb_n[2] + d])
                    s2 = s2 + du * n2
                    n3 = plsc.load_gather(n_buf, [pn[3], cb_n[3] + d])
                    s3 = s3 + du * n3
                    n4 = plsc.load_gather(n_buf, [pn[4], cb_n[4] + d])
                    s4 = s4 + du * n4
                    return (sp, s0, s1, s2, s3, s4)

                z = jnp.zeros((16,), jnp.float32)
                sp, s0, s1, s2, s3, s4 = lax.fori_loop(
                    0, _D, d_body, (z, z, z, z, z, z))
                val = _softplus(-jnp.clip(sp, -10.0, 10.0))
                for sk in (s0, s1, s2, s3, s4):
                    val = val + _softplus(jnp.clip(sk, -10.0, 10.0))
                return acc + val

            return lax.fori_loop(0, _NG, group_body, acc)

        acc = lax.fori_loop(0, _NCH, chunk_body, jnp.zeros((16,), jnp.float32))
        accv[...] = acc
        pltpu.sync_copy(accv, out_hbm.at[wid])

    return k(pos_u, pos_v, neg_t, u_pairs, v_pairs)


_HV = _VOCAB // 2
_RW_BLK = 2048


def _repack(w_t):
    def body(in_ref, o_ref):
        x = in_ref[...]
        o_ref[:, 0:64] = x[:, :_RW_BLK].T
        o_ref[:, 64:128] = x[:, _RW_BLK:].T

    nj = pl.cdiv(_HV, _RW_BLK)
    return pl.pallas_call(
        body,
        grid=(nj,),
        in_specs=[pl.BlockSpec((64, 2 * _RW_BLK), lambda j: (0, j))],
        out_specs=pl.BlockSpec((_RW_BLK, 128), lambda j: (j, 0)),
        out_shape=jax.ShapeDtypeStruct((_HV, 128), jnp.float32),
    )(w_t)


def _finalize(partials):
    def body(p_ref, o_ref):
        o_ref[0, 0] = jnp.sum(p_ref[...]) * (1.0 / _B)

    out = pl.pallas_call(
        body,
        out_shape=jax.ShapeDtypeStruct((1, 1), jnp.float32),
        out_specs=pl.BlockSpec(memory_space=pltpu.SMEM),
    )(partials)
    return out[0, 0]


def kernel(pos_u, pos_v, neg_v, u_weight, v_weight):
    u_pairs = _repack(u_weight.T)
    v_pairs = _repack(v_weight.T)
    neg_t = neg_v.astype(jnp.int32).T
    partials = _sc_partials(pos_u.astype(jnp.int32), pos_v.astype(jnp.int32),
                            neg_t, u_pairs, v_pairs)
    return _finalize(partials)
